# baseline (speedup 1.0000x reference)
import functools

import jax
import jax.numpy as jnp
from jax import lax
from jax.experimental import pallas as pl
from jax.experimental.pallas import tpu as pltpu
from jax.experimental.pallas import tpu_sc as plsc

NODE = 512
FEAT = 512
PAIRS = 16
NWORK = 32
HALF = NODE // 2
CHUNK = 64
NCH = HALF // CHUNK
ABLK = NODE + 128
SCALE = 1.0 / (512.0 ** 0.5)


@functools.lru_cache(maxsize=None)
def _sc_gather_fn(total):
    info = plsc.get_sparse_core_info()
    nc = info.num_cores

    @functools.partial(
        pl.kernel,
        mesh=plsc.VectorSubcoreMesh(core_axis_name="c", subcore_axis_name="s"),
        out_type=jax.ShapeDtypeStruct((PAIRS * NODE, FEAT), jnp.float32),
        scratch_types=[
            pltpu.VMEM((2 * NWORK, 16), jnp.int32),
            pltpu.VMEM((CHUNK, FEAT), jnp.float32),
            pltpu.VMEM((CHUNK, FEAT), jnp.float32),
            pltpu.SemaphoreType.DMA,
            pltpu.SemaphoreType.DMA,
        ],
    )
    def gather(features_hbm, meta_hbm, out_hbm, meta_v, buf0, buf1, s0, s1):
        wid = lax.axis_index("s") * nc + lax.axis_index("c")
        pltpu.sync_copy(meta_hbm, meta_v)
        lane = lax.iota(jnp.int32, 16)
        start_w = meta_v[wid][0]
        cnt_w = meta_v[wid + NWORK][0]
        nch = (cnt_w + (CHUNK - 1)) // CHUNK

        bufs = (buf0, buf1)
        sems = (s0, s1)

        def make_issue(j):
            def _():
                for t in range(CHUNK // 16):
                    ridx = jnp.minimum(
                        start_w + (j * CHUNK + t * 16) + lane, total - 1)
                    pltpu.async_copy(
                        features_hbm.at[ridx],
                        bufs[j % 2].at[pl.ds(t * 16, 16)],
                        sems[j % 2])
            return _

        def make_retire(j):
            def _():
                pltpu.make_async_copy(
                    features_hbm.at[pl.ds(0, CHUNK)],
                    bufs[j % 2], sems[j % 2]).wait()
                pltpu.sync_copy(
                    bufs[j % 2],
                    out_hbm.at[pl.ds(wid * HALF + j * CHUNK, CHUNK)])
            return _

        for j in range(NCH):
            pl.when(j < nch)(make_issue(j))
            if j > 0:
                pl.when(j - 1 < nch)(make_retire(j - 1))
        pl.when(NCH - 1 < nch)(make_retire(NCH - 1))

    return gather


def _tc_body(meta_ref, src_ref, feat_ref, out_ref, rbuf, sem):
    b = pl.program_id(0)
    s = meta_ref[b, 0]
    r = meta_ref[b, 1]

    def issue(i, slot):
        astart = pl.multiple_of(meta_ref[i, 2], 8)
        pltpu.make_async_copy(
            feat_ref.at[pl.ds(astart, ABLK)],
            rbuf.at[slot, pl.ds(0, ABLK)], sem.at[slot]).start()

    def wait(slot):
        pltpu.make_async_copy(
            feat_ref.at[pl.ds(0, ABLK)],
            rbuf.at[slot, pl.ds(0, ABLK)], sem.at[slot]).wait()

    @pl.when(b == 0)
    def _():
        issue(0, 0)

    @pl.when(b + 1 < PAIRS)
    def _():
        issue(b + 1, (b + 1) % 2)

    wait(b % 2)
    delta = meta_ref[b, 3]
    ld = rbuf[b % 2]
    ref_blk = pltpu.roll(ld, ABLK - delta, axis=0)[:NODE]

    acc = lax.dot_general(
        src_ref[0], ref_blk,
        (((1,), (1,)), ((), ())),
        preferred_element_type=jnp.float32,
    )
    rows = lax.broadcasted_iota(jnp.int32, (NODE, NODE), 0)
    cols = lax.broadcasted_iota(jnp.int32, (NODE, NODE), 1)
    mask = (rows < s) & (cols < r)
    out_ref[0] = jnp.where(mask, acc * SCALE, 0.0)


_tc_scores = pl.pallas_call(
    _tc_body,
    grid=(PAIRS,),
    in_specs=[
        pl.BlockSpec(memory_space=pltpu.SMEM),
        pl.BlockSpec((1, NODE, FEAT), lambda b: (b, 0, 0)),
        pl.BlockSpec(memory_space=pl.ANY),
    ],
    out_specs=pl.BlockSpec((1, NODE, NODE), lambda b: (b, 0, 0)),
    out_shape=jax.ShapeDtypeStruct((PAIRS, NODE, NODE), jnp.float32),
    scratch_shapes=[
        pltpu.VMEM((2, ABLK, FEAT), jnp.float32),
        pltpu.SemaphoreType.DMA((2,)),
    ],
)


def kernel(features, src_ref_counts):
    total = features.shape[0]
    counts = jnp.asarray(src_ref_counts).astype(jnp.int32)
    s = counts[:, 0]
    r = counts[:, 1]
    tot = s + r
    starts = jnp.cumsum(tot) - tot

    halves = jnp.tile(jnp.asarray([0, HALF], jnp.int32), PAIRS)
    w_start = jnp.repeat(starts, 2) + halves
    half_cnt = jnp.clip(
        jnp.repeat(jnp.minimum(s, NODE), 2) - halves, 0, HALF)
    sc_meta = jnp.broadcast_to(
        jnp.concatenate([w_start, half_cnt])[:, None],
        (2 * NWORK, 16)).astype(jnp.int32)

    rstart = starts + s
    astart = jnp.clip((rstart // 8) * 8, 0, total - ABLK)
    delta = rstart - astart
    tc_meta = jnp.stack([s, r, astart, delta], axis=1)

    gathered = _sc_gather_fn(total)(features, sc_meta)
    blocks = gathered.reshape(PAIRS, NODE, FEAT)
    return _tc_scores(tc_meta, blocks, features)


# speedup vs baseline: 1.0417x; 1.0417x over previous
import functools

import jax
import jax.numpy as jnp
from jax import lax
from jax.experimental import pallas as pl
from jax.experimental.pallas import tpu as pltpu
from jax.experimental.pallas import tpu_sc as plsc

NODE = 512
FEAT = 512
PAIRS = 16
NWORK = 32
HALF = NODE // 2
CHUNK = 64
NCH = HALF // CHUNK
ABLK = NODE + 8
SCALE = 1.0 / (512.0 ** 0.5)


@functools.lru_cache(maxsize=None)
def _sc_gather_fn(total):
    info = plsc.get_sparse_core_info()
    nc = info.num_cores

    @functools.partial(
        pl.kernel,
        mesh=plsc.VectorSubcoreMesh(core_axis_name="c", subcore_axis_name="s"),
        out_type=jax.ShapeDtypeStruct((PAIRS * NODE, FEAT), jnp.float32),
        scratch_types=[
            pltpu.VMEM((2 * NWORK, 16), jnp.int32),
            pltpu.VMEM((CHUNK, FEAT), jnp.float32),
            pltpu.VMEM((CHUNK, FEAT), jnp.float32),
            pltpu.SemaphoreType.DMA,
            pltpu.SemaphoreType.DMA,
        ],
    )
    def gather(features_hbm, meta_hbm, out_hbm, meta_v, buf0, buf1, s0, s1):
        wid = lax.axis_index("s") * nc + lax.axis_index("c")
        pltpu.sync_copy(meta_hbm, meta_v)
        lane = lax.iota(jnp.int32, 16)
        start_w = meta_v[wid][0]
        cnt_w = meta_v[wid + NWORK][0]
        nch = (cnt_w + (CHUNK - 1)) // CHUNK

        bufs = (buf0, buf1)
        sems = (s0, s1)

        def make_issue(j):
            def _():
                for t in range(CHUNK // 16):
                    ridx = jnp.minimum(
                        start_w + (j * CHUNK + t * 16) + lane, total - 1)
                    pltpu.async_copy(
                        features_hbm.at[ridx],
                        bufs[j % 2].at[pl.ds(t * 16, 16)],
                        sems[j % 2])
            return _

        def make_retire(j):
            def _():
                pltpu.make_async_copy(
                    features_hbm.at[pl.ds(0, CHUNK)],
                    bufs[j % 2], sems[j % 2]).wait()
                pltpu.sync_copy(
                    bufs[j % 2],
                    out_hbm.at[pl.ds(wid * HALF + j * CHUNK, CHUNK)])
            return _

        for j in range(NCH):
            pl.when(j < nch)(make_issue(j))
            if j > 0:
                pl.when(j - 1 < nch)(make_retire(j - 1))
        pl.when(NCH - 1 < nch)(make_retire(NCH - 1))

    return gather


def _tc_body(meta_ref, src_ref, feat_ref, out_ref, rbuf, sem):
    b = pl.program_id(0)
    s = meta_ref[b, 0]
    r = meta_ref[b, 1]

    def issue(i, slot):
        astart = pl.multiple_of(meta_ref[i, 2], 8)
        pltpu.make_async_copy(
            feat_ref.at[pl.ds(astart, ABLK)],
            rbuf.at[slot, pl.ds(0, ABLK)], sem.at[slot]).start()

    def wait(slot):
        pltpu.make_async_copy(
            feat_ref.at[pl.ds(0, ABLK)],
            rbuf.at[slot, pl.ds(0, ABLK)], sem.at[slot]).wait()

    @pl.when(b == 0)
    def _():
        issue(0, 0)

    @pl.when(b + 1 < PAIRS)
    def _():
        issue(b + 1, (b + 1) % 2)

    wait(b % 2)
    delta = meta_ref[b, 3]
    ld = rbuf[b % 2]
    ref_blk = pltpu.roll(ld, ABLK - delta, axis=0)[:NODE]

    acc = lax.dot_general(
        src_ref[0], ref_blk,
        (((1,), (1,)), ((), ())),
        preferred_element_type=jnp.float32,
    )
    rows = lax.broadcasted_iota(jnp.int32, (NODE, NODE), 0)
    cols = lax.broadcasted_iota(jnp.int32, (NODE, NODE), 1)
    mask = (rows < s) & (cols < r)
    out_ref[0] = jnp.where(mask, acc * SCALE, 0.0)


_tc_scores = pl.pallas_call(
    _tc_body,
    grid=(PAIRS,),
    in_specs=[
        pl.BlockSpec(memory_space=pltpu.SMEM),
        pl.BlockSpec((1, NODE, FEAT), lambda b: (b, 0, 0)),
        pl.BlockSpec(memory_space=pl.ANY),
    ],
    out_specs=pl.BlockSpec((1, NODE, NODE), lambda b: (b, 0, 0)),
    out_shape=jax.ShapeDtypeStruct((PAIRS, NODE, NODE), jnp.float32),
    scratch_shapes=[
        pltpu.VMEM((2, ABLK, FEAT), jnp.float32),
        pltpu.SemaphoreType.DMA((2,)),
    ],
)


def kernel(features, src_ref_counts):
    total = features.shape[0]
    counts = jnp.asarray(src_ref_counts).astype(jnp.int32)
    s = counts[:, 0]
    r = counts[:, 1]
    tot = s + r
    starts = jnp.cumsum(tot) - tot

    halves = jnp.tile(jnp.asarray([0, HALF], jnp.int32), PAIRS)
    w_start = jnp.repeat(starts, 2) + halves
    half_cnt = jnp.clip(
        jnp.repeat(jnp.minimum(s, NODE), 2) - halves, 0, HALF)
    sc_meta = jnp.broadcast_to(
        jnp.concatenate([w_start, half_cnt])[:, None],
        (2 * NWORK, 16)).astype(jnp.int32)

    rstart = starts + s
    astart = jnp.clip((rstart // 8) * 8, 0, total - ABLK)
    delta = rstart - astart
    tc_meta = jnp.stack([s, r, astart, delta], axis=1)

    gathered = _sc_gather_fn(total)(features, sc_meta)
    blocks = gathered.reshape(PAIRS, NODE, FEAT)
    return _tc_scores(tc_meta, blocks, features)
